# baseline (speedup 1.0000x reference)
import functools

import jax
import jax.numpy as jnp
from jax import lax
from jax.experimental import pallas as pl
from jax.experimental.pallas import tpu as pltpu
from jax.experimental.pallas import tpu_sc as plsc

VOCAB = 1000000
EMB = 64
B = 16384
L = 50
NC = 2
NS = 16
NW = NC * NS
TOT = B * L
PER_W = TOT // NW
CHUNK = 128
NCHUNK = PER_W // CHUNK

_mesh = plsc.VectorSubcoreMesh(
    core_axis_name="c", subcore_axis_name="s", num_cores=NC, num_subcores=NS
)


def _detile_body(x_ref, o_ref):
    y = x_ref[...].T
    for t in range(4):
        o_ref[64 * t : 64 * t + 64, 0:64] = y[128 * t : 128 * t + 64, :]
        o_ref[64 * t : 64 * t + 64, 64:128] = y[128 * t + 64 : 128 * t + 128, :]


_DT_COLS = 512
_DT_GRID = (VOCAB + _DT_COLS - 1) // _DT_COLS
_ST_ROWS = ((VOCAB + 127) // 128) * 64

_detile = pl.pallas_call(
    _detile_body,
    grid=(_DT_GRID,),
    in_specs=[pl.BlockSpec((EMB, _DT_COLS), lambda i: (0, i))],
    out_specs=pl.BlockSpec((_DT_COLS // 2, 128), lambda i: (i, 0)),
    out_shape=jax.ShapeDtypeStruct((_ST_ROWS, 128), jnp.float32),
)


def _tile5_body(x_ref, o_ref):
    ft = x_ref[...].T
    for k in range(8):
        ok = jnp.concatenate(
            [ft[0:64, 64 * k : 64 * k + 64], ft[64:128, 64 * k : 64 * k + 64]],
            axis=1,
        )
        o_ref[0, :, k, :, :] = ok.reshape(8, 8, 128)


_tile5 = pl.pallas_call(
    _tile5_body,
    grid=(L, B // 1024),
    in_specs=[pl.BlockSpec((512, 128), lambda l, c: (l * 16 + c, 0))],
    out_specs=pl.BlockSpec((1, 8, 8, 8, 128), lambda l, c: (l, 0, c, 0, 0)),
    out_shape=jax.ShapeDtypeStruct((L, 8, B // 128, 8, 128), jnp.float32),
)


@functools.partial(
    pl.kernel,
    out_type=jax.ShapeDtypeStruct((TOT // 2, 128), jnp.float32),
    mesh=_mesh,
    compiler_params=pltpu.CompilerParams(use_tc_tiling_on_sc=False),
    scratch_types=[
        pltpu.VMEM((NCHUNK, CHUNK), jnp.int32),
        pltpu.VMEM((2, CHUNK, EMB), jnp.float32),
        pltpu.SemaphoreType.DMA((2,)),
        pltpu.SemaphoreType.DMA((2,)),
    ],
)
def _gather(idx_hbm, table_hbm, out_hbm, idx_v, rows_v, gsem, osem):
    wid = lax.axis_index("s") * NC + lax.axis_index("c")
    base2 = wid * (PER_W // 2)
    pltpu.sync_copy(idx_hbm.at[wid], idx_v)

    def gather(j, slot):
        return pltpu.make_async_copy(
            table_hbm.at[idx_v.at[j]], rows_v.at[slot], gsem.at[slot]
        )

    def out_copies(j, slot):
        fr0 = base2 + j * (CHUNK // 2)
        return (
            pltpu.make_async_copy(
                rows_v.at[slot, pl.ds(0, 64)],
                out_hbm.at[pl.ds(fr0, 64), pl.ds(0, 64)],
                osem.at[slot],
            ),
            pltpu.make_async_copy(
                rows_v.at[slot, pl.ds(64, 64)],
                out_hbm.at[pl.ds(fr0, 64), pl.ds(64, 64)],
                osem.at[slot],
            ),
        )

    def out_start(j, slot):
        for c in out_copies(j, slot):
            c.start()

    def out_wait(j, slot):
        for c in out_copies(j, slot):
            c.wait()

    gather(0, 0).start()

    @pl.loop(0, NCHUNK, step=2)
    def _(i):
        for b in range(2):
            j = i + b

            @pl.when(j >= 1)
            def _():
                out_wait(j - 1, 1 - b)

            @pl.when(j + 1 < NCHUNK)
            def _():
                gather(j + 1, 1 - b).start()

            gather(j, b).wait()
            out_start(j, b)

    out_wait(NCHUNK - 1, (NCHUNK - 1) % 2)


def kernel(input, table):
    idx = input.T.reshape(NW, NCHUNK, CHUNK)
    vm = idx % 128
    idx_g = (idx // 128) * 128 + jnp.where(vm < 64, 2 * vm, 2 * vm - 127)
    table_st = _detile(table.T)
    out_st = _gather(idx_g, table_st.reshape(2 * _ST_ROWS, EMB))
    out5 = _tile5(out_st)
    return out5.transpose(2, 4, 0, 1, 3).reshape(B, L, EMB)


# speedup vs baseline: 2.5586x; 2.5586x over previous
import functools

import jax
import jax.numpy as jnp
from jax import lax
from jax.experimental import pallas as pl
from jax.experimental.pallas import tpu as pltpu
from jax.experimental.pallas import tpu_sc as plsc

VOCAB = 1000000
EMB = 64
B = 16384
L = 50
NC = 2
NS = 16
NW = NC * NS
TOT = B * L
PER_W = TOT // NW
CHUNK = 128
NCHUNK = PER_W // CHUNK

_mesh = plsc.VectorSubcoreMesh(
    core_axis_name="c", subcore_axis_name="s", num_cores=NC, num_subcores=NS
)


def _detile_body(x_ref, o_ref):
    y = x_ref[...].T
    for t in range(_DT_COLS // 128):
        o_ref[64 * t : 64 * t + 64, 0:64] = y[128 * t : 128 * t + 64, :]
        o_ref[64 * t : 64 * t + 64, 64:128] = y[128 * t + 64 : 128 * t + 128, :]


_DT_COLS = 4096
_DT_GRID = (VOCAB + _DT_COLS - 1) // _DT_COLS
_ST_ROWS = ((VOCAB + 127) // 128) * 64

_detile = pl.pallas_call(
    _detile_body,
    grid=(_DT_GRID,),
    in_specs=[pl.BlockSpec((EMB, _DT_COLS), lambda i: (0, i))],
    out_specs=pl.BlockSpec((_DT_COLS // 2, 128), lambda i: (i, 0)),
    out_shape=jax.ShapeDtypeStruct((_ST_ROWS, 128), jnp.float32),
)


_T5_ROWS = 2048


def _tile5_body(x_ref, o_ref):
    ft = x_ref[...].T
    for k in range(_T5_ROWS // 64):
        ok = jnp.concatenate(
            [ft[0:64, 64 * k : 64 * k + 64], ft[64:128, 64 * k : 64 * k + 64]],
            axis=1,
        )
        o_ref[0, :, k, :, :] = ok.reshape(8, 8, 128)


_tile5 = pl.pallas_call(
    _tile5_body,
    grid=(L, B // (2 * _T5_ROWS)),
    in_specs=[pl.BlockSpec((_T5_ROWS, 128), lambda l, c: (l * (8192 // _T5_ROWS) + c, 0))],
    out_specs=pl.BlockSpec(
        (1, 8, _T5_ROWS // 64, 8, 128), lambda l, c: (l, 0, c, 0, 0)
    ),
    out_shape=jax.ShapeDtypeStruct((L, 8, B // 128, 8, 128), jnp.float32),
)


@functools.partial(
    pl.kernel,
    out_type=jax.ShapeDtypeStruct((TOT // 2, 128), jnp.float32),
    mesh=_mesh,
    compiler_params=pltpu.CompilerParams(use_tc_tiling_on_sc=False),
    scratch_types=[
        pltpu.VMEM((NCHUNK, CHUNK), jnp.int32),
        pltpu.VMEM((2, CHUNK, EMB), jnp.float32),
        pltpu.SemaphoreType.DMA((2,)),
        pltpu.SemaphoreType.DMA((2,)),
    ],
)
def _gather(idx_hbm, table_hbm, out_hbm, idx_v, rows_v, gsem, osem):
    wid = lax.axis_index("s") * NC + lax.axis_index("c")
    base2 = wid * (PER_W // 2)
    pltpu.sync_copy(idx_hbm.at[wid], idx_v)

    def gather(j, slot):
        return pltpu.make_async_copy(
            table_hbm.at[idx_v.at[j]], rows_v.at[slot], gsem.at[slot]
        )

    def out_copies(j, slot):
        fr0 = base2 + j * (CHUNK // 2)
        return (
            pltpu.make_async_copy(
                rows_v.at[slot, pl.ds(0, 64)],
                out_hbm.at[pl.ds(fr0, 64), pl.ds(0, 64)],
                osem.at[slot],
            ),
            pltpu.make_async_copy(
                rows_v.at[slot, pl.ds(64, 64)],
                out_hbm.at[pl.ds(fr0, 64), pl.ds(64, 64)],
                osem.at[slot],
            ),
        )

    def out_start(j, slot):
        for c in out_copies(j, slot):
            c.start()

    def out_wait(j, slot):
        for c in out_copies(j, slot):
            c.wait()

    gather(0, 0).start()

    @pl.loop(0, NCHUNK, step=2)
    def _(i):
        for b in range(2):
            j = i + b

            @pl.when(j >= 1)
            def _():
                out_wait(j - 1, 1 - b)

            @pl.when(j + 1 < NCHUNK)
            def _():
                gather(j + 1, 1 - b).start()

            gather(j, b).wait()
            out_start(j, b)

    out_wait(NCHUNK - 1, (NCHUNK - 1) % 2)


def kernel(input, table):
    idx = input.T.reshape(NW, NCHUNK, CHUNK)
    vm = idx % 128
    idx_g = (idx // 128) * 128 + jnp.where(vm < 64, 2 * vm, 2 * vm - 127)
    table_st = _detile(table.T)
    out_st = _gather(idx_g, table_st.reshape(2 * _ST_ROWS, EMB))
    out5 = _tile5(out_st)
    return out5.transpose(2, 4, 0, 1, 3).reshape(B, L, EMB)


# speedup vs baseline: 3.0623x; 1.1969x over previous
import functools

import jax
import jax.numpy as jnp
from jax import lax
from jax.experimental import pallas as pl
from jax.experimental.pallas import tpu as pltpu
from jax.experimental.pallas import tpu_sc as plsc

VOCAB = 1000000
EMB = 64
B = 16384
L = 50
NC = 2
NS = 16
NW = NC * NS
TOT = B * L
PER_W = TOT // NW
CHUNK = 128
NCHUNK = PER_W // CHUNK

_mesh = plsc.VectorSubcoreMesh(
    core_axis_name="c", subcore_axis_name="s", num_cores=NC, num_subcores=NS
)


def _detile_body(x_ref, o_ref):
    y = x_ref[...].T
    for t in range(_DT_COLS // 128):
        o_ref[64 * t : 64 * t + 64, 0:64] = y[128 * t : 128 * t + 64, :]
        o_ref[64 * t : 64 * t + 64, 64:128] = y[128 * t + 64 : 128 * t + 128, :]


_DT_COLS = 8192
_DT_GRID = (VOCAB + _DT_COLS - 1) // _DT_COLS
_ST_ROWS = ((VOCAB + 127) // 128) * 64

_detile = pl.pallas_call(
    _detile_body,
    grid=(_DT_GRID,),
    in_specs=[pl.BlockSpec((EMB, _DT_COLS), lambda i: (0, i))],
    out_specs=pl.BlockSpec((_DT_COLS // 2, 128), lambda i: (i, 0)),
    out_shape=jax.ShapeDtypeStruct((_ST_ROWS, 128), jnp.float32),
)


_T5_ROWS = 4096


def _tile5_body(x_ref, o_ref):
    ft = x_ref[...].T
    for k in range(_T5_ROWS // 64):
        ok = jnp.concatenate(
            [ft[0:64, 64 * k : 64 * k + 64], ft[64:128, 64 * k : 64 * k + 64]],
            axis=1,
        )
        o_ref[0, :, k, :, :] = ok.reshape(8, 8, 128)


_tile5 = pl.pallas_call(
    _tile5_body,
    grid=(L, B // (2 * _T5_ROWS)),
    in_specs=[pl.BlockSpec((_T5_ROWS, 128), lambda l, c: (l * (8192 // _T5_ROWS) + c, 0))],
    out_specs=pl.BlockSpec(
        (1, 8, _T5_ROWS // 64, 8, 128), lambda l, c: (l, 0, c, 0, 0)
    ),
    out_shape=jax.ShapeDtypeStruct((L, 8, B // 128, 8, 128), jnp.float32),
)


@functools.partial(
    pl.kernel,
    out_type=jax.ShapeDtypeStruct((TOT // 2, 128), jnp.float32),
    mesh=_mesh,
    compiler_params=pltpu.CompilerParams(use_tc_tiling_on_sc=False),
    scratch_types=[
        pltpu.VMEM((NCHUNK, CHUNK), jnp.int32),
        pltpu.VMEM((2, CHUNK, EMB), jnp.float32),
        pltpu.SemaphoreType.DMA((2,)),
        pltpu.SemaphoreType.DMA((2,)),
    ],
)
def _gather(idx_hbm, table_hbm, out_hbm, idx_v, rows_v, gsem, osem):
    wid = lax.axis_index("s") * NC + lax.axis_index("c")
    base2 = wid * (PER_W // 2)
    pltpu.sync_copy(idx_hbm.at[wid], idx_v)

    def gather(j, slot):
        return pltpu.make_async_copy(
            table_hbm.at[idx_v.at[j]], rows_v.at[slot], gsem.at[slot]
        )

    def out_copies(j, slot):
        fr0 = base2 + j * (CHUNK // 2)
        return (
            pltpu.make_async_copy(
                rows_v.at[slot, pl.ds(0, 64)],
                out_hbm.at[pl.ds(fr0, 64), pl.ds(0, 64)],
                osem.at[slot],
            ),
            pltpu.make_async_copy(
                rows_v.at[slot, pl.ds(64, 64)],
                out_hbm.at[pl.ds(fr0, 64), pl.ds(64, 64)],
                osem.at[slot],
            ),
        )

    def out_start(j, slot):
        for c in out_copies(j, slot):
            c.start()

    def out_wait(j, slot):
        for c in out_copies(j, slot):
            c.wait()

    gather(0, 0).start()

    @pl.loop(0, NCHUNK, step=2)
    def _(i):
        for b in range(2):
            j = i + b

            @pl.when(j >= 1)
            def _():
                out_wait(j - 1, 1 - b)

            @pl.when(j + 1 < NCHUNK)
            def _():
                gather(j + 1, 1 - b).start()

            gather(j, b).wait()
            out_start(j, b)

    out_wait(NCHUNK - 1, (NCHUNK - 1) % 2)


def kernel(input, table):
    idx = input.T.reshape(NW, NCHUNK, CHUNK)
    vm = idx % 128
    idx_g = (idx // 128) * 128 + jnp.where(vm < 64, 2 * vm, 2 * vm - 127)
    table_st = _detile(table.T)
    out_st = _gather(idx_g, table_st.reshape(2 * _ST_ROWS, EMB))
    out5 = _tile5(out_st)
    return out5.transpose(2, 4, 0, 1, 3).reshape(B, L, EMB)


# speedup vs baseline: 3.4080x; 1.1129x over previous
import functools

import jax
import jax.numpy as jnp
from jax import lax
from jax.experimental import pallas as pl
from jax.experimental.pallas import tpu as pltpu
from jax.experimental.pallas import tpu_sc as plsc

VOCAB = 1000000
EMB = 64
B = 16384
L = 50
NC = 2
NS = 16
NW = NC * NS
TOT = B * L
PER_W = TOT // NW
CHUNK = 128
NCHUNK = PER_W // CHUNK

_mesh = plsc.VectorSubcoreMesh(
    core_axis_name="c", subcore_axis_name="s", num_cores=NC, num_subcores=NS
)


def _detile_body(x_ref, o_ref):
    y = x_ref[...].T
    for t in range(_DT_COLS // 128):
        o_ref[64 * t : 64 * t + 64, 0:64] = y[128 * t : 128 * t + 64, :]
        o_ref[64 * t : 64 * t + 64, 64:128] = y[128 * t + 64 : 128 * t + 128, :]


_DT_COLS = 16384
_DT_GRID = (VOCAB + _DT_COLS - 1) // _DT_COLS
_ST_ROWS = ((VOCAB + 127) // 128) * 64

_detile = pl.pallas_call(
    _detile_body,
    grid=(_DT_GRID,),
    in_specs=[pl.BlockSpec((EMB, _DT_COLS), lambda i: (0, i))],
    out_specs=pl.BlockSpec((_DT_COLS // 2, 128), lambda i: (i, 0)),
    out_shape=jax.ShapeDtypeStruct((_ST_ROWS, 128), jnp.float32),
)


_T5_ROWS = 8192


def _tile5_body(x_ref, o_ref):
    ft = x_ref[...].T
    for k in range(_T5_ROWS // 64):
        ok = jnp.concatenate(
            [ft[0:64, 64 * k : 64 * k + 64], ft[64:128, 64 * k : 64 * k + 64]],
            axis=1,
        )
        o_ref[0, :, k, :, :] = ok.reshape(8, 8, 128)


_tile5 = pl.pallas_call(
    _tile5_body,
    grid=(L, B // (2 * _T5_ROWS)),
    in_specs=[pl.BlockSpec((_T5_ROWS, 128), lambda l, c: (l * (8192 // _T5_ROWS) + c, 0))],
    out_specs=pl.BlockSpec(
        (1, 8, _T5_ROWS // 64, 8, 128), lambda l, c: (l, 0, c, 0, 0)
    ),
    out_shape=jax.ShapeDtypeStruct((L, 8, B // 128, 8, 128), jnp.float32),
)


@functools.partial(
    pl.kernel,
    out_type=jax.ShapeDtypeStruct((TOT // 2, 128), jnp.float32),
    mesh=_mesh,
    compiler_params=pltpu.CompilerParams(use_tc_tiling_on_sc=False),
    scratch_types=[
        pltpu.VMEM((NCHUNK, CHUNK), jnp.int32),
        pltpu.VMEM((2, CHUNK, EMB), jnp.float32),
        pltpu.SemaphoreType.DMA((2,)),
        pltpu.SemaphoreType.DMA((2,)),
    ],
)
def _gather(idx_hbm, table_hbm, out_hbm, idx_v, rows_v, gsem, osem):
    wid = lax.axis_index("s") * NC + lax.axis_index("c")
    base2 = wid * (PER_W // 2)
    pltpu.sync_copy(idx_hbm.at[wid], idx_v)

    def gather(j, slot):
        return pltpu.make_async_copy(
            table_hbm.at[idx_v.at[j]], rows_v.at[slot], gsem.at[slot]
        )

    def out_copies(j, slot):
        fr0 = base2 + j * (CHUNK // 2)
        return (
            pltpu.make_async_copy(
                rows_v.at[slot, pl.ds(0, 64)],
                out_hbm.at[pl.ds(fr0, 64), pl.ds(0, 64)],
                osem.at[slot],
            ),
            pltpu.make_async_copy(
                rows_v.at[slot, pl.ds(64, 64)],
                out_hbm.at[pl.ds(fr0, 64), pl.ds(64, 64)],
                osem.at[slot],
            ),
        )

    def out_start(j, slot):
        for c in out_copies(j, slot):
            c.start()

    def out_wait(j, slot):
        for c in out_copies(j, slot):
            c.wait()

    gather(0, 0).start()

    @pl.loop(0, NCHUNK, step=2)
    def _(i):
        for b in range(2):
            j = i + b

            @pl.when(j >= 1)
            def _():
                out_wait(j - 1, 1 - b)

            @pl.when(j + 1 < NCHUNK)
            def _():
                gather(j + 1, 1 - b).start()

            gather(j, b).wait()
            out_start(j, b)

    out_wait(NCHUNK - 1, (NCHUNK - 1) % 2)


def kernel(input, table):
    idx = input.T.reshape(NW, NCHUNK, CHUNK)
    vm = idx % 128
    idx_g = (idx // 128) * 128 + jnp.where(vm < 64, 2 * vm, 2 * vm - 127)
    table_st = _detile(table.T)
    out_st = _gather(idx_g, table_st.reshape(2 * _ST_ROWS, EMB))
    out5 = _tile5(out_st)
    return out5.transpose(2, 4, 0, 1, 3).reshape(B, L, EMB)


# speedup vs baseline: 3.5796x; 1.0503x over previous
import functools

import jax
import jax.numpy as jnp
from jax import lax
from jax.experimental import pallas as pl
from jax.experimental.pallas import tpu as pltpu
from jax.experimental.pallas import tpu_sc as plsc

VOCAB = 1000000
EMB = 64
B = 16384
L = 50
NC = 2
NS = 16
NW = NC * NS
TOT = B * L
PER_W = TOT // NW
CHUNK = 128
NCHUNK = PER_W // CHUNK

_mesh = plsc.VectorSubcoreMesh(
    core_axis_name="c", subcore_axis_name="s", num_cores=NC, num_subcores=NS
)


def _detile_body(x_ref, o_ref):
    y = x_ref[...].T
    for t in range(_DT_COLS // 128):
        o_ref[64 * t : 64 * t + 64, 0:64] = y[128 * t : 128 * t + 64, :]
        o_ref[64 * t : 64 * t + 64, 64:128] = y[128 * t + 64 : 128 * t + 128, :]


_DT_COLS = 32768
_DT_GRID = (VOCAB + _DT_COLS - 1) // _DT_COLS
_ST_ROWS = ((VOCAB + 127) // 128) * 64

_detile = pl.pallas_call(
    _detile_body,
    grid=(_DT_GRID,),
    in_specs=[pl.BlockSpec((EMB, _DT_COLS), lambda i: (0, i))],
    out_specs=pl.BlockSpec((_DT_COLS // 2, 128), lambda i: (i, 0)),
    out_shape=jax.ShapeDtypeStruct((_ST_ROWS, 128), jnp.float32),
)


_T5_ROWS = 8192


_T5_LS = 2


def _tile5_body(x_ref, o_ref):
    ft = x_ref[...].T
    for q in range(_T5_LS):
        for k in range(128):
            c0 = q * 8192 + 64 * k
            ok = jnp.concatenate(
                [ft[0:64, c0 : c0 + 64], ft[64:128, c0 : c0 + 64]], axis=1
            )
            o_ref[q, :, k, :, :] = ok.reshape(8, 8, 128)


_tile5 = pl.pallas_call(
    _tile5_body,
    grid=(L // _T5_LS,),
    in_specs=[pl.BlockSpec((_T5_LS * 8192, 128), lambda i: (i, 0))],
    out_specs=pl.BlockSpec((_T5_LS, 8, 128, 8, 128), lambda i: (i, 0, 0, 0, 0)),
    out_shape=jax.ShapeDtypeStruct((L, 8, B // 128, 8, 128), jnp.float32),
)


@functools.partial(
    pl.kernel,
    out_type=jax.ShapeDtypeStruct((TOT // 2, 128), jnp.float32),
    mesh=_mesh,
    compiler_params=pltpu.CompilerParams(use_tc_tiling_on_sc=False),
    scratch_types=[
        pltpu.VMEM((NCHUNK, CHUNK), jnp.int32),
        pltpu.VMEM((2, CHUNK, EMB), jnp.float32),
        pltpu.SemaphoreType.DMA((2,)),
        pltpu.SemaphoreType.DMA((2,)),
    ],
)
def _gather(idx_hbm, table_hbm, out_hbm, idx_v, rows_v, gsem, osem):
    wid = lax.axis_index("s") * NC + lax.axis_index("c")
    base2 = wid * (PER_W // 2)
    pltpu.sync_copy(idx_hbm.at[wid], idx_v)

    def gather(j, slot):
        return pltpu.make_async_copy(
            table_hbm.at[idx_v.at[j]], rows_v.at[slot], gsem.at[slot]
        )

    def out_copies(j, slot):
        fr0 = base2 + j * (CHUNK // 2)
        return (
            pltpu.make_async_copy(
                rows_v.at[slot, pl.ds(0, 64)],
                out_hbm.at[pl.ds(fr0, 64), pl.ds(0, 64)],
                osem.at[slot],
            ),
            pltpu.make_async_copy(
                rows_v.at[slot, pl.ds(64, 64)],
                out_hbm.at[pl.ds(fr0, 64), pl.ds(64, 64)],
                osem.at[slot],
            ),
        )

    def out_start(j, slot):
        for c in out_copies(j, slot):
            c.start()

    def out_wait(j, slot):
        for c in out_copies(j, slot):
            c.wait()

    gather(0, 0).start()

    @pl.loop(0, NCHUNK, step=2)
    def _(i):
        for b in range(2):
            j = i + b

            @pl.when(j >= 1)
            def _():
                out_wait(j - 1, 1 - b)

            @pl.when(j + 1 < NCHUNK)
            def _():
                gather(j + 1, 1 - b).start()

            gather(j, b).wait()
            out_start(j, b)

    out_wait(NCHUNK - 1, (NCHUNK - 1) % 2)


def kernel(input, table):
    idx = input.T.reshape(NW, NCHUNK, CHUNK)
    vm = idx % 128
    idx_g = (idx // 128) * 128 + jnp.where(vm < 64, 2 * vm, 2 * vm - 127)
    table_st = _detile(table.T)
    out_st = _gather(idx_g, table_st.reshape(2 * _ST_ROWS, EMB))
    out5 = _tile5(out_st)
    return out5.transpose(2, 4, 0, 1, 3).reshape(B, L, EMB)
